# baseline (speedup 1.0000x reference)
import functools

import jax
import jax.numpy as jnp
from jax import lax
from jax.experimental import pallas as pl
from jax.experimental.pallas import tpu as pltpu
from jax.experimental.pallas import tpu_sc as plsc

B = 2
N = 4096
M = 4096
K0, K1 = 16, 32
R0SQ, R1SQ = 0.1 * 0.1, 0.2 * 0.2
FDIM, HDIM, ODIM = 64, 128, 256

NSUB = 16
NQ = N // NSUB
LANES = 16

_mesh = plsc.VectorSubcoreMesh(core_axis_name="c", subcore_axis_name="s",
                               num_cores=2, num_subcores=16)


def _round_bf16(x):
    u = plsc.bitcast(x, jnp.int32)
    bias = 0x7FFF + (lax.shift_right_logical(u, 16) & 1)
    return plsc.bitcast((u + bias) & jnp.int32(-65536), jnp.float32)


@functools.partial(
    pl.kernel,
    out_type=(
        jax.ShapeDtypeStruct((B, NSUB, NQ * K0 * 4), jnp.float32),
        jax.ShapeDtypeStruct((B, NSUB, NQ * K1 * 4), jnp.float32),
    ),
    mesh=_mesh,
    compiler_params=pltpu.CompilerParams(needs_layout_passes=False),
    scratch_types=[
        pltpu.VMEM((M,), jnp.float32),
        pltpu.VMEM((M,), jnp.float32),
        pltpu.VMEM((M,), jnp.float32),
        pltpu.VMEM((M,), jnp.float32),
        pltpu.VMEM((M,), jnp.float32),
        pltpu.VMEM((M,), jnp.float32),
        pltpu.VMEM((M,), jnp.float32),
        pltpu.VMEM((NQ + LANES,), jnp.float32),
        pltpu.VMEM((NQ + LANES,), jnp.float32),
        pltpu.VMEM((NQ + LANES,), jnp.float32),
        pltpu.VMEM((NQ + LANES,), jnp.float32),
        pltpu.VMEM((NQ + LANES,), jnp.float32),
        pltpu.VMEM((NQ + LANES,), jnp.float32),
        pltpu.VMEM((NQ + LANES,), jnp.float32),
        pltpu.VMEM((NQ * K0 * 4 + LANES,), jnp.float32),
        pltpu.VMEM((NQ * K1 * 4 + LANES,), jnp.float32),
    ],
)
def _ball_query_sc(qx_h, qy_h, qz_h, kx_h, ky_h, kz_h, out0_h, out1_h,
                   kxb, kyb, kzb, kxr, kyr, kzr, kkb,
                   qxb, qyb, qzb, qxr, qyr, qzr, qqb, f0b, f1b):
    b = lax.axis_index("c")
    s = lax.axis_index("s")
    pltpu.sync_copy(kx_h.at[b], kxb)
    pltpu.sync_copy(ky_h.at[b], kyb)
    pltpu.sync_copy(kz_h.at[b], kzb)
    pltpu.sync_copy(qx_h.at[b, s], qxb.at[pl.ds(0, NQ)])
    pltpu.sync_copy(qy_h.at[b, s], qyb.at[pl.ds(0, NQ)])
    pltpu.sync_copy(qz_h.at[b, s], qzb.at[pl.ds(0, NQ)])

    iota = lax.iota(jnp.int32, LANES)

    def ksetup(v, _):
        off = v * LANES
        kxv = kxb[pl.ds(off, LANES)]
        kyv = kyb[pl.ds(off, LANES)]
        kzv = kzb[pl.ds(off, LANES)]
        kxr[pl.ds(off, LANES)] = _round_bf16(kxv)
        kyr[pl.ds(off, LANES)] = _round_bf16(kyv)
        kzr[pl.ds(off, LANES)] = _round_bf16(kzv)
        kkb[pl.ds(off, LANES)] = kxv * kxv + kyv * kyv + kzv * kzv
        return 0

    lax.fori_loop(0, M // LANES, ksetup, 0)

    def qsetup(v, _):
        off = v * LANES
        qxv = qxb[pl.ds(off, LANES)]
        qyv = qyb[pl.ds(off, LANES)]
        qzv = qzb[pl.ds(off, LANES)]
        qxr[pl.ds(off, LANES)] = _round_bf16(qxv)
        qyr[pl.ds(off, LANES)] = _round_bf16(qyv)
        qzr[pl.ds(off, LANES)] = _round_bf16(qzv)
        qqb[pl.ds(off, LANES)] = qxv * qxv + qyv * qyv + qzv * qzv
        return 0

    lax.fori_loop(0, NQ // LANES, qsetup, 0)

    kx0 = kxb[pl.ds(0, LANES)][0]
    ky0 = kyb[pl.ds(0, LANES)][0]
    kz0 = kzb[pl.ds(0, LANES)][0]

    def qbody(n, _):
        qxn = qxb[pl.ds(n, LANES)][0]
        qyn = qyb[pl.ds(n, LANES)][0]
        qzn = qzb[pl.ds(n, LANES)][0]
        bqx = qxr[pl.ds(n, LANES)][0]
        bqy = qyr[pl.ds(n, LANES)][0]
        bqz = qzr[pl.ds(n, LANES)][0]
        qqn = qqb[pl.ds(n, LANES)][0]
        fb0 = n * (K0 * 4)
        fb1 = n * (K1 * 4)

        def vbody(v, carry):
            c0v, c1v = carry
            off = v * LANES
            dot = (bqx * kxr[pl.ds(off, LANES)]
                   + bqy * kyr[pl.ds(off, LANES)]
                   + bqz * kzr[pl.ds(off, LANES)])
            d2m = (qqn + kkb[pl.ds(off, LANES)]) - 2.0 * dot
            m1 = d2m <= R1SQ
            pc1 = plsc.all_reduce_population_count(m1)

            def hit1(c0v, c1v):
                dx = kxb[pl.ds(off, LANES)] - qxn
                dy = kyb[pl.ds(off, LANES)] - qyn
                dz = kzb[pl.ds(off, LANES)] - qzn
                d2 = dx * dx + dy * dy + dz * dz
                r1 = plsc.cumsum(m1.astype(jnp.int32))
                pos1 = c1v + r1 - 1
                keep1 = m1 & (pos1 < K1)
                idx1 = jnp.minimum(pos1, K1 - 1) * 4 + fb1
                plsc.store_scatter(f1b, [idx1], dx, mask=keep1)
                plsc.store_scatter(f1b, [idx1 + 1], dy, mask=keep1)
                plsc.store_scatter(f1b, [idx1 + 2], dz, mask=keep1)
                plsc.store_scatter(f1b, [idx1 + 3], d2, mask=keep1)

                m0 = d2m <= R0SQ
                pc0 = plsc.all_reduce_population_count(m0)

                def hit0(c0v):
                    r0 = plsc.cumsum(m0.astype(jnp.int32))
                    pos0 = c0v + r0 - 1
                    keep0 = m0 & (pos0 < K0)
                    idx0 = jnp.minimum(pos0, K0 - 1) * 4 + fb0
                    plsc.store_scatter(f0b, [idx0], dx, mask=keep0)
                    plsc.store_scatter(f0b, [idx0 + 1], dy, mask=keep0)
                    plsc.store_scatter(f0b, [idx0 + 2], dz, mask=keep0)
                    plsc.store_scatter(f0b, [idx0 + 3], d2, mask=keep0)
                    return c0v + pc0

                c0v = lax.cond(pc0[0] > 0, hit0, lambda c: c, c0v)
                return c0v, c1v + pc1

            return lax.cond(pc1[0] > 0, hit1,
                            lambda a, b_: (a, b_), c0v, c1v)

        zero = jnp.zeros((LANES,), jnp.int32)
        c0v, c1v = lax.fori_loop(0, M // LANES, vbody, (zero, zero))

        dx0 = kx0 - qxn
        dy0 = ky0 - qyn
        dz0 = kz0 - qzn
        d20 = dx0 * dx0 + dy0 * dy0 + dz0 * dz0
        cnt0 = jnp.max(c0v)
        cnt1 = jnp.max(c1v)
        for c, init in ((0, dx0), (1, dy0), (2, dz0), (3, d20)):
            v0 = jnp.where(cnt0 > 0, f0b[pl.ds(fb0 + c, LANES)][0], init)
            plsc.store_scatter(f0b, [iota * 4 + (fb0 + c)],
                               jnp.broadcast_to(v0, (LANES,)),
                               mask=iota >= c0v)
            v1 = jnp.where(cnt1 > 0, f1b[pl.ds(fb1 + c, LANES)][0], init)
            plsc.store_scatter(f1b, [iota * 4 + (fb1 + c)],
                               jnp.broadcast_to(v1, (LANES,)),
                               mask=iota >= c1v)
            plsc.store_scatter(f1b, [iota * 4 + (fb1 + 64 + c)],
                               jnp.broadcast_to(v1, (LANES,)),
                               mask=(iota + LANES) >= c1v)
        return 0

    lax.fori_loop(0, NQ, qbody, 0)
    pltpu.sync_copy(f0b.at[pl.ds(0, NQ * K0 * 4)], out0_h.at[b, s])
    pltpu.sync_copy(f1b.at[pl.ds(0, NQ * K1 * 4)], out1_h.at[b, s])


_TC_ROWS = 256


def _mlp_body(f0_ref, f1_ref, w10_ref, b10_ref, w20_ref, b20_ref,
              w11_ref, b11_ref, w21_ref, b21_ref, wp_ref, bp_ref, out_ref):
    def branch(f_ref, w1_ref, b1_ref, w2_ref, b2_ref, k):
        f = f_ref[0]
        dist = jnp.sqrt(f[:, 3:4] + 1e-12)
        feat = jnp.concatenate([f[:, 0:3], dist], axis=1)
        h = jnp.dot(feat, w1_ref[...], preferred_element_type=jnp.float32)
        h = jnp.maximum(h + b1_ref[...], 0.0)
        h = jnp.dot(h, w2_ref[...], preferred_element_type=jnp.float32)
        h = jnp.maximum(h + b2_ref[...], 0.0)
        return jnp.max(h.reshape(_TC_ROWS, k, HDIM), axis=1)

    g0 = branch(f0_ref, w10_ref, b10_ref, w20_ref, b20_ref, K0)
    g1 = branch(f1_ref, w11_ref, b11_ref, w21_ref, b21_ref, K1)
    g = jnp.concatenate([g0, g1], axis=1)
    out = jnp.dot(g, wp_ref[...], preferred_element_type=jnp.float32)
    out_ref[0] = out + bp_ref[...]


def _mlp_tc(f0, f1, w10, b10, w20, b20, w11, b11, w21, b21, wp, bp):
    grid = (B, N // _TC_ROWS)
    full = lambda r, c: pl.BlockSpec((r, c), lambda i, j: (0, 0))
    return pl.pallas_call(
        _mlp_body,
        grid=grid,
        in_specs=[
            pl.BlockSpec((1, _TC_ROWS * K0, 4), lambda i, j: (i, j, 0)),
            pl.BlockSpec((1, _TC_ROWS * K1, 4), lambda i, j: (i, j, 0)),
            full(4, FDIM), full(1, FDIM),
            full(FDIM, HDIM), full(1, HDIM),
            full(4, FDIM), full(1, FDIM),
            full(FDIM, HDIM), full(1, HDIM),
            full(2 * HDIM, ODIM), full(1, ODIM),
        ],
        out_specs=pl.BlockSpec((1, _TC_ROWS, ODIM), lambda i, j: (i, j, 0)),
        out_shape=jax.ShapeDtypeStruct((B, N, ODIM), jnp.float32),
    )(f0, f1, w10, b10, w20, b20, w11, b11, w21, b21, wp, bp)


def kernel(query_points, key_points, W1_0, b1_0, W2_0, b2_0,
           W1_1, b1_1, W2_1, b2_1, Wp, bp):
    qx = query_points[:, :, 0].reshape(B, NSUB, NQ)
    qy = query_points[:, :, 1].reshape(B, NSUB, NQ)
    qz = query_points[:, :, 2].reshape(B, NSUB, NQ)
    kx = key_points[:, :, 0]
    ky = key_points[:, :, 1]
    kz = key_points[:, :, 2]

    out0, out1 = _ball_query_sc(qx, qy, qz, kx, ky, kz)
    f0 = out0.reshape(B, N * K0, 4)
    f1 = out1.reshape(B, N * K1, 4)

    return _mlp_tc(f0, f1, W1_0, b1_0.reshape(1, FDIM), W2_0,
                   b2_0.reshape(1, HDIM), W1_1, b1_1.reshape(1, FDIM),
                   W2_1, b2_1.reshape(1, HDIM), Wp, bp.reshape(1, ODIM))


# speedup vs baseline: 2.5982x; 2.5982x over previous
import functools

import jax
import jax.numpy as jnp
from jax import lax
from jax.experimental import pallas as pl
from jax.experimental.pallas import tpu as pltpu
from jax.experimental.pallas import tpu_sc as plsc

B = 2
N = 4096
M = 4096
K0, K1 = 16, 32
R0SQ, R1SQ = 0.1 * 0.1, 0.2 * 0.2
FDIM, HDIM, ODIM = 64, 128, 256

NSUB = 16
NQ = N // NSUB
LANES = 16

_mesh = plsc.VectorSubcoreMesh(core_axis_name="c", subcore_axis_name="s",
                               num_cores=2, num_subcores=16)


def _round_bf16(x):
    u = plsc.bitcast(x, jnp.int32)
    bias = 0x7FFF + (lax.shift_right_logical(u, 16) & 1)
    return plsc.bitcast((u + bias) & jnp.int32(-65536), jnp.float32)


@functools.partial(
    pl.kernel,
    out_type=(
        jax.ShapeDtypeStruct((B, NSUB, NQ * K0 * 4), jnp.float32),
        jax.ShapeDtypeStruct((B, NSUB, NQ * K1 * 4), jnp.float32),
    ),
    mesh=_mesh,
    compiler_params=pltpu.CompilerParams(needs_layout_passes=False),
    scratch_types=[
        pltpu.VMEM((M,), jnp.float32),
        pltpu.VMEM((M,), jnp.float32),
        pltpu.VMEM((M,), jnp.float32),
        pltpu.VMEM((M,), jnp.float32),
        pltpu.VMEM((M,), jnp.float32),
        pltpu.VMEM((M,), jnp.float32),
        pltpu.VMEM((M,), jnp.float32),
        pltpu.VMEM((NQ + LANES,), jnp.float32),
        pltpu.VMEM((NQ + LANES,), jnp.float32),
        pltpu.VMEM((NQ + LANES,), jnp.float32),
        pltpu.VMEM((NQ + LANES,), jnp.float32),
        pltpu.VMEM((NQ + LANES,), jnp.float32),
        pltpu.VMEM((NQ + LANES,), jnp.float32),
        pltpu.VMEM((NQ + LANES,), jnp.float32),
        pltpu.VMEM((NQ * K0 * 4 + LANES,), jnp.float32),
        pltpu.VMEM((NQ * K1 * 4 + LANES,), jnp.float32),
    ],
)
def _ball_query_sc(qx_h, qy_h, qz_h, kx_h, ky_h, kz_h, out0_h, out1_h,
                   kxb, kyb, kzb, kxr, kyr, kzr, kkb,
                   qxb, qyb, qzb, qxr, qyr, qzr, qqb, f0b, f1b):
    b = lax.axis_index("c")
    s = lax.axis_index("s")
    pltpu.sync_copy(kx_h.at[b], kxb)
    pltpu.sync_copy(ky_h.at[b], kyb)
    pltpu.sync_copy(kz_h.at[b], kzb)
    pltpu.sync_copy(qx_h.at[b, s], qxb.at[pl.ds(0, NQ)])
    pltpu.sync_copy(qy_h.at[b, s], qyb.at[pl.ds(0, NQ)])
    pltpu.sync_copy(qz_h.at[b, s], qzb.at[pl.ds(0, NQ)])

    iota = lax.iota(jnp.int32, LANES)

    def ksetup(v, _):
        off = v * LANES
        kxv = kxb[pl.ds(off, LANES)]
        kyv = kyb[pl.ds(off, LANES)]
        kzv = kzb[pl.ds(off, LANES)]
        kxr[pl.ds(off, LANES)] = _round_bf16(kxv)
        kyr[pl.ds(off, LANES)] = _round_bf16(kyv)
        kzr[pl.ds(off, LANES)] = _round_bf16(kzv)
        kkb[pl.ds(off, LANES)] = kxv * kxv + kyv * kyv + kzv * kzv
        return 0

    lax.fori_loop(0, M // LANES, ksetup, 0)

    def qsetup(v, _):
        off = v * LANES
        qxv = qxb[pl.ds(off, LANES)]
        qyv = qyb[pl.ds(off, LANES)]
        qzv = qzb[pl.ds(off, LANES)]
        qxr[pl.ds(off, LANES)] = _round_bf16(qxv)
        qyr[pl.ds(off, LANES)] = _round_bf16(qyv)
        qzr[pl.ds(off, LANES)] = _round_bf16(qzv)
        qqb[pl.ds(off, LANES)] = qxv * qxv + qyv * qyv + qzv * qzv
        return 0

    lax.fori_loop(0, NQ // LANES, qsetup, 0)

    kx0 = kxb[pl.ds(0, LANES)][0]
    ky0 = kyb[pl.ds(0, LANES)][0]
    kz0 = kzb[pl.ds(0, LANES)][0]

    def qbody(n, _):
        qxn = qxb[pl.ds(n, LANES)][0]
        qyn = qyb[pl.ds(n, LANES)][0]
        qzn = qzb[pl.ds(n, LANES)][0]
        bqx = qxr[pl.ds(n, LANES)][0]
        bqy = qyr[pl.ds(n, LANES)][0]
        bqz = qzr[pl.ds(n, LANES)][0]
        qqn = qqb[pl.ds(n, LANES)][0]
        fb0 = n * (K0 * 4)
        fb1 = n * (K1 * 4)

        zero = jnp.zeros((LANES,), jnp.int32)

        @plsc.parallel_loop(0, M // LANES, unroll=4, carry=(zero, zero))
        def vloop(v, carry):
            c0v, c1v = carry
            off = v * LANES
            dx = kxb[pl.ds(off, LANES)] - qxn
            dy = kyb[pl.ds(off, LANES)] - qyn
            dz = kzb[pl.ds(off, LANES)] - qzn
            d2 = dx * dx + dy * dy + dz * dz
            dot = (bqx * kxr[pl.ds(off, LANES)]
                   + bqy * kyr[pl.ds(off, LANES)]
                   + bqz * kzr[pl.ds(off, LANES)])
            d2m = (qqn + kkb[pl.ds(off, LANES)]) - 2.0 * dot

            m0 = d2m <= R0SQ
            r0 = plsc.cumsum(m0.astype(jnp.int32))
            pos0 = c0v + r0 - 1
            keep0 = m0 & (pos0 < K0)
            idx0 = jnp.minimum(pos0, K0 - 1) * 4 + fb0
            plsc.store_scatter(f0b, [idx0], dx, mask=keep0)
            plsc.store_scatter(f0b, [idx0 + 1], dy, mask=keep0)
            plsc.store_scatter(f0b, [idx0 + 2], dz, mask=keep0)
            plsc.store_scatter(f0b, [idx0 + 3], d2, mask=keep0)

            m1 = d2m <= R1SQ
            r1 = plsc.cumsum(m1.astype(jnp.int32))
            pos1 = c1v + r1 - 1
            keep1 = m1 & (pos1 < K1)
            idx1 = jnp.minimum(pos1, K1 - 1) * 4 + fb1
            plsc.store_scatter(f1b, [idx1], dx, mask=keep1)
            plsc.store_scatter(f1b, [idx1 + 1], dy, mask=keep1)
            plsc.store_scatter(f1b, [idx1 + 2], dz, mask=keep1)
            plsc.store_scatter(f1b, [idx1 + 3], d2, mask=keep1)

            return (c0v + plsc.all_reduce_population_count(m0),
                    c1v + plsc.all_reduce_population_count(m1))

        c0v, c1v = vloop

        dx0 = kx0 - qxn
        dy0 = ky0 - qyn
        dz0 = kz0 - qzn
        d20 = dx0 * dx0 + dy0 * dy0 + dz0 * dz0
        cnt0 = jnp.max(c0v)
        cnt1 = jnp.max(c1v)
        for c, init in ((0, dx0), (1, dy0), (2, dz0), (3, d20)):
            v0 = jnp.where(cnt0 > 0, f0b[pl.ds(fb0 + c, LANES)][0], init)
            plsc.store_scatter(f0b, [iota * 4 + (fb0 + c)],
                               jnp.broadcast_to(v0, (LANES,)),
                               mask=iota >= c0v)
            v1 = jnp.where(cnt1 > 0, f1b[pl.ds(fb1 + c, LANES)][0], init)
            plsc.store_scatter(f1b, [iota * 4 + (fb1 + c)],
                               jnp.broadcast_to(v1, (LANES,)),
                               mask=iota >= c1v)
            plsc.store_scatter(f1b, [iota * 4 + (fb1 + 64 + c)],
                               jnp.broadcast_to(v1, (LANES,)),
                               mask=(iota + LANES) >= c1v)
        return 0

    lax.fori_loop(0, NQ, qbody, 0)
    pltpu.sync_copy(f0b.at[pl.ds(0, NQ * K0 * 4)], out0_h.at[b, s])
    pltpu.sync_copy(f1b.at[pl.ds(0, NQ * K1 * 4)], out1_h.at[b, s])


_TC_ROWS = 256


def _mlp_body(f0_ref, f1_ref, w10_ref, b10_ref, w20_ref, b20_ref,
              w11_ref, b11_ref, w21_ref, b21_ref, wp_ref, bp_ref, out_ref):
    def branch(f_ref, w1_ref, b1_ref, w2_ref, b2_ref, k):
        f = f_ref[0]
        dist = jnp.sqrt(f[:, 3:4] + 1e-12)
        feat = jnp.concatenate([f[:, 0:3], dist], axis=1)
        h = jnp.dot(feat, w1_ref[...], preferred_element_type=jnp.float32)
        h = jnp.maximum(h + b1_ref[...], 0.0)
        h = jnp.dot(h, w2_ref[...], preferred_element_type=jnp.float32)
        h = jnp.maximum(h + b2_ref[...], 0.0)
        return jnp.max(h.reshape(_TC_ROWS, k, HDIM), axis=1)

    g0 = branch(f0_ref, w10_ref, b10_ref, w20_ref, b20_ref, K0)
    g1 = branch(f1_ref, w11_ref, b11_ref, w21_ref, b21_ref, K1)
    g = jnp.concatenate([g0, g1], axis=1)
    out = jnp.dot(g, wp_ref[...], preferred_element_type=jnp.float32)
    out_ref[0] = out + bp_ref[...]


def _mlp_tc(f0, f1, w10, b10, w20, b20, w11, b11, w21, b21, wp, bp):
    grid = (B, N // _TC_ROWS)
    full = lambda r, c: pl.BlockSpec((r, c), lambda i, j: (0, 0))
    return pl.pallas_call(
        _mlp_body,
        grid=grid,
        in_specs=[
            pl.BlockSpec((1, _TC_ROWS * K0, 4), lambda i, j: (i, j, 0)),
            pl.BlockSpec((1, _TC_ROWS * K1, 4), lambda i, j: (i, j, 0)),
            full(4, FDIM), full(1, FDIM),
            full(FDIM, HDIM), full(1, HDIM),
            full(4, FDIM), full(1, FDIM),
            full(FDIM, HDIM), full(1, HDIM),
            full(2 * HDIM, ODIM), full(1, ODIM),
        ],
        out_specs=pl.BlockSpec((1, _TC_ROWS, ODIM), lambda i, j: (i, j, 0)),
        out_shape=jax.ShapeDtypeStruct((B, N, ODIM), jnp.float32),
    )(f0, f1, w10, b10, w20, b20, w11, b11, w21, b21, wp, bp)


def kernel(query_points, key_points, W1_0, b1_0, W2_0, b2_0,
           W1_1, b1_1, W2_1, b2_1, Wp, bp):
    qx = query_points[:, :, 0].reshape(B, NSUB, NQ)
    qy = query_points[:, :, 1].reshape(B, NSUB, NQ)
    qz = query_points[:, :, 2].reshape(B, NSUB, NQ)
    kx = key_points[:, :, 0]
    ky = key_points[:, :, 1]
    kz = key_points[:, :, 2]

    out0, out1 = _ball_query_sc(qx, qy, qz, kx, ky, kz)
    f0 = out0.reshape(B, N * K0, 4)
    f1 = out1.reshape(B, N * K1, 4)

    return _mlp_tc(f0, f1, W1_0, b1_0.reshape(1, FDIM), W2_0,
                   b2_0.reshape(1, HDIM), W1_1, b1_1.reshape(1, FDIM),
                   W2_1, b2_1.reshape(1, HDIM), Wp, bp.reshape(1, ODIM))


# speedup vs baseline: 3.6429x; 1.4021x over previous
import functools

import jax
import jax.numpy as jnp
from jax import lax
from jax.experimental import pallas as pl
from jax.experimental.pallas import tpu as pltpu
from jax.experimental.pallas import tpu_sc as plsc

B = 2
N = 4096
M = 4096
K0, K1 = 16, 32
R0SQ, R1SQ = 0.1 * 0.1, 0.2 * 0.2
FDIM, HDIM, ODIM = 64, 128, 256

NSUB = 16
NQ = N // NSUB
LANES = 16

_mesh = plsc.VectorSubcoreMesh(core_axis_name="c", subcore_axis_name="s",
                               num_cores=2, num_subcores=16)


def _round_bf16(x):
    u = plsc.bitcast(x, jnp.int32)
    bias = 0x7FFF + (lax.shift_right_logical(u, 16) & 1)
    return plsc.bitcast((u + bias) & jnp.int32(-65536), jnp.float32)


@functools.partial(
    pl.kernel,
    out_type=(
        jax.ShapeDtypeStruct((B, NSUB, NQ * K0 * 4), jnp.float32),
        jax.ShapeDtypeStruct((B, NSUB, NQ * K1 * 4), jnp.float32),
    ),
    mesh=_mesh,
    compiler_params=pltpu.CompilerParams(needs_layout_passes=False),
    scratch_types=[
        pltpu.VMEM((M,), jnp.float32),
        pltpu.VMEM((M,), jnp.float32),
        pltpu.VMEM((M,), jnp.float32),
        pltpu.VMEM((M,), jnp.float32),
        pltpu.VMEM((M,), jnp.float32),
        pltpu.VMEM((M,), jnp.float32),
        pltpu.VMEM((M,), jnp.float32),
        pltpu.VMEM((NQ + LANES,), jnp.float32),
        pltpu.VMEM((NQ + LANES,), jnp.float32),
        pltpu.VMEM((NQ + LANES,), jnp.float32),
        pltpu.VMEM((NQ + LANES,), jnp.float32),
        pltpu.VMEM((NQ + LANES,), jnp.float32),
        pltpu.VMEM((NQ + LANES,), jnp.float32),
        pltpu.VMEM((NQ + LANES,), jnp.float32),
        pltpu.VMEM((NQ * K0 * 4 + LANES,), jnp.float32),
        pltpu.VMEM((NQ * K1 * 4 + LANES,), jnp.float32),
        pltpu.VMEM((NQ * K0 + LANES,), jnp.int32),
        pltpu.VMEM((NQ * K1 + LANES,), jnp.int32),
    ],
)
def _ball_query_sc(qx_h, qy_h, qz_h, kx_h, ky_h, kz_h, out0_h, out1_h,
                   kxb, kyb, kzb, kxr, kyr, kzr, kkb,
                   qxb, qyb, qzb, qxr, qyr, qzr, qqb, f0b, f1b, i0b, i1b):
    b = lax.axis_index("c")
    s = lax.axis_index("s")
    pltpu.sync_copy(kx_h.at[b], kxb)
    pltpu.sync_copy(ky_h.at[b], kyb)
    pltpu.sync_copy(kz_h.at[b], kzb)
    pltpu.sync_copy(qx_h.at[b, s], qxb.at[pl.ds(0, NQ)])
    pltpu.sync_copy(qy_h.at[b, s], qyb.at[pl.ds(0, NQ)])
    pltpu.sync_copy(qz_h.at[b, s], qzb.at[pl.ds(0, NQ)])

    iota = lax.iota(jnp.int32, LANES)

    def ksetup(v, _):
        off = v * LANES
        kxv = kxb[pl.ds(off, LANES)]
        kyv = kyb[pl.ds(off, LANES)]
        kzv = kzb[pl.ds(off, LANES)]
        kxr[pl.ds(off, LANES)] = _round_bf16(kxv)
        kyr[pl.ds(off, LANES)] = _round_bf16(kyv)
        kzr[pl.ds(off, LANES)] = _round_bf16(kzv)
        kkb[pl.ds(off, LANES)] = kxv * kxv + kyv * kyv + kzv * kzv
        return 0

    lax.fori_loop(0, M // LANES, ksetup, 0)

    def qsetup(v, _):
        off = v * LANES
        qxv = qxb[pl.ds(off, LANES)]
        qyv = qyb[pl.ds(off, LANES)]
        qzv = qzb[pl.ds(off, LANES)]
        qxr[pl.ds(off, LANES)] = _round_bf16(qxv)
        qyr[pl.ds(off, LANES)] = _round_bf16(qyv)
        qzr[pl.ds(off, LANES)] = _round_bf16(qzv)
        qqb[pl.ds(off, LANES)] = qxv * qxv + qyv * qyv + qzv * qzv
        return 0

    lax.fori_loop(0, NQ // LANES, qsetup, 0)

    zerov = jnp.zeros((LANES,), jnp.int32)

    def qbody(n, _):
        qxn = qxb[pl.ds(n, LANES)][0]
        qyn = qyb[pl.ds(n, LANES)][0]
        qzn = qzb[pl.ds(n, LANES)][0]
        bqx = qxr[pl.ds(n, LANES)][0]
        bqy = qyr[pl.ds(n, LANES)][0]
        bqz = qzr[pl.ds(n, LANES)][0]
        qqn = qqb[pl.ds(n, LANES)][0]
        ib0 = n * K0
        ib1 = n * K1
        fb0 = n * (K0 * 4)
        fb1 = n * (K1 * 4)

        cb0z = jnp.broadcast_to(ib0 - 1, (LANES,))
        cb1z = jnp.broadcast_to(ib1 - 1, (LANES,))

        @plsc.parallel_loop(0, M // LANES, unroll=4, carry=(cb0z, cb1z))
        def vloop(v, carry):
            cb0, cb1 = carry
            off = v * LANES
            dot = (bqx * kxr[pl.ds(off, LANES)]
                   + bqy * kyr[pl.ds(off, LANES)]
                   + bqz * kzr[pl.ds(off, LANES)])
            d2m = (qqn + kkb[pl.ds(off, LANES)]) - 2.0 * dot
            keyidx = iota + off
            m0 = d2m <= R0SQ
            m1 = d2m <= R1SQ
            r0 = plsc.cumsum(m0.astype(jnp.int32))
            r1 = plsc.cumsum(m1.astype(jnp.int32))
            p0 = cb0 + r0
            p1 = cb1 + r1
            keep0 = m0 & (p0 < ib0 + K0)
            keep1 = m1 & (p1 < ib1 + K1)
            plsc.store_scatter(i0b, [jnp.minimum(p0, ib0 + K0 - 1)],
                               keyidx, mask=keep0)
            plsc.store_scatter(i1b, [jnp.minimum(p1, ib1 + K1 - 1)],
                               keyidx, mask=keep1)
            return (cb0 + plsc.all_reduce_population_count(m0),
                    cb1 + plsc.all_reduce_population_count(m1))

        cb0, cb1 = vloop
        cnt0v = cb0 - (ib0 - 1)
        cnt1v = cb1 - (ib1 - 1)

        iv0 = i0b[pl.ds(ib0, LANES)]
        pad0 = jnp.where(cnt0v > 0, jnp.broadcast_to(iv0[0], (LANES,)), zerov)
        idxs0 = jnp.where(iota < cnt0v, iv0, pad0)
        gx = plsc.load_gather(kxb, [idxs0])
        gy = plsc.load_gather(kyb, [idxs0])
        gz = plsc.load_gather(kzb, [idxs0])
        dx = gx - qxn
        dy = gy - qyn
        dz = gz - qzn
        d2 = dx * dx + dy * dy + dz * dz
        base0 = iota * 4 + fb0
        plsc.store_scatter(f0b, [base0], dx)
        plsc.store_scatter(f0b, [base0 + 1], dy)
        plsc.store_scatter(f0b, [base0 + 2], dz)
        plsc.store_scatter(f0b, [base0 + 3], d2)

        iv1a = i1b[pl.ds(ib1, LANES)]
        pad1 = jnp.where(cnt1v > 0, jnp.broadcast_to(iv1a[0], (LANES,)), zerov)
        for g in range(K1 // LANES):
            ivg = i1b[pl.ds(ib1 + g * LANES, LANES)] if g else iv1a
            slot = iota + g * LANES
            idxs1 = jnp.where(slot < cnt1v, ivg, pad1)
            gx = plsc.load_gather(kxb, [idxs1])
            gy = plsc.load_gather(kyb, [idxs1])
            gz = plsc.load_gather(kzb, [idxs1])
            dx = gx - qxn
            dy = gy - qyn
            dz = gz - qzn
            d2 = dx * dx + dy * dy + dz * dz
            base1 = slot * 4 + fb1
            plsc.store_scatter(f1b, [base1], dx)
            plsc.store_scatter(f1b, [base1 + 1], dy)
            plsc.store_scatter(f1b, [base1 + 2], dz)
            plsc.store_scatter(f1b, [base1 + 3], d2)
        return 0

    lax.fori_loop(0, NQ, qbody, 0)
    pltpu.sync_copy(f0b.at[pl.ds(0, NQ * K0 * 4)], out0_h.at[b, s])
    pltpu.sync_copy(f1b.at[pl.ds(0, NQ * K1 * 4)], out1_h.at[b, s])


_TC_ROWS = 256


def _mlp_body(f0_ref, f1_ref, w10_ref, b10_ref, w20_ref, b20_ref,
              w11_ref, b11_ref, w21_ref, b21_ref, wp_ref, bp_ref, out_ref):
    def branch(f_ref, w1_ref, b1_ref, w2_ref, b2_ref, k):
        f = f_ref[0]
        dist = jnp.sqrt(f[:, 3:4] + 1e-12)
        feat = jnp.concatenate([f[:, 0:3], dist], axis=1)
        h = jnp.dot(feat, w1_ref[...], preferred_element_type=jnp.float32)
        h = jnp.maximum(h + b1_ref[...], 0.0)
        h = jnp.dot(h, w2_ref[...], preferred_element_type=jnp.float32)
        h = jnp.maximum(h + b2_ref[...], 0.0)
        return jnp.max(h.reshape(_TC_ROWS, k, HDIM), axis=1)

    g0 = branch(f0_ref, w10_ref, b10_ref, w20_ref, b20_ref, K0)
    g1 = branch(f1_ref, w11_ref, b11_ref, w21_ref, b21_ref, K1)
    g = jnp.concatenate([g0, g1], axis=1)
    out = jnp.dot(g, wp_ref[...], preferred_element_type=jnp.float32)
    out_ref[0] = out + bp_ref[...]


def _mlp_tc(f0, f1, w10, b10, w20, b20, w11, b11, w21, b21, wp, bp):
    grid = (B, N // _TC_ROWS)
    full = lambda r, c: pl.BlockSpec((r, c), lambda i, j: (0, 0))
    return pl.pallas_call(
        _mlp_body,
        grid=grid,
        in_specs=[
            pl.BlockSpec((1, _TC_ROWS * K0, 4), lambda i, j: (i, j, 0)),
            pl.BlockSpec((1, _TC_ROWS * K1, 4), lambda i, j: (i, j, 0)),
            full(4, FDIM), full(1, FDIM),
            full(FDIM, HDIM), full(1, HDIM),
            full(4, FDIM), full(1, FDIM),
            full(FDIM, HDIM), full(1, HDIM),
            full(2 * HDIM, ODIM), full(1, ODIM),
        ],
        out_specs=pl.BlockSpec((1, _TC_ROWS, ODIM), lambda i, j: (i, j, 0)),
        out_shape=jax.ShapeDtypeStruct((B, N, ODIM), jnp.float32),
    )(f0, f1, w10, b10, w20, b20, w11, b11, w21, b21, wp, bp)


def kernel(query_points, key_points, W1_0, b1_0, W2_0, b2_0,
           W1_1, b1_1, W2_1, b2_1, Wp, bp):
    qx = query_points[:, :, 0].reshape(B, NSUB, NQ)
    qy = query_points[:, :, 1].reshape(B, NSUB, NQ)
    qz = query_points[:, :, 2].reshape(B, NSUB, NQ)
    kx = key_points[:, :, 0]
    ky = key_points[:, :, 1]
    kz = key_points[:, :, 2]

    out0, out1 = _ball_query_sc(qx, qy, qz, kx, ky, kz)
    f0 = out0.reshape(B, N * K0, 4)
    f1 = out1.reshape(B, N * K1, 4)

    return _mlp_tc(f0, f1, W1_0, b1_0.reshape(1, FDIM), W2_0,
                   b2_0.reshape(1, HDIM), W1_1, b1_1.reshape(1, FDIM),
                   W2_1, b2_1.reshape(1, HDIM), Wp, bp.reshape(1, ODIM))
